# baseline (speedup 1.0000x reference)
import jax
import jax.numpy as jnp
from jax import lax
from jax.experimental import pallas as pl
from jax.experimental.pallas import tpu as pltpu
from jax.experimental.pallas import tpu_sc as plsc

D = 768
V = 1024
B = 16384
NC = 2
NS = 16
NW = NC * NS
B_PER_W = B // NW
CHUNK = 16
NBUF = 8
NCHUNK = B_PER_W // CHUNK


def _gather_body(table_hbm, idx_hbm, out_hbm, idx_v, rows_v, gsems, ssems):
    cid = lax.axis_index("c")
    sid = lax.axis_index("s")
    wid = sid * NC + cid
    base = wid * B_PER_W

    pltpu.sync_copy(idx_hbm.at[pl.ds(base, B_PER_W)], idx_v)

    def gather(c):
        return pltpu.async_copy(
            table_hbm.at[idx_v.at[pl.ds(c * CHUNK, CHUNK)]],
            rows_v.at[c % NBUF],
            gsems.at[c % NBUF],
        )

    def store(c):
        return pltpu.async_copy(
            rows_v.at[c % NBUF],
            out_hbm.at[pl.ds(base + c * CHUNK, CHUNK)],
            ssems.at[c % NBUF],
        )

    skew = NBUF // 2
    gcp = [None] * NCHUNK
    scp = [None] * NCHUNK
    for i in range(skew):
        gcp[i] = gather(i)
    for i in range(NCHUNK):
        if i >= skew:
            scp[i - skew].wait()
        nxt = i + skew
        if nxt < NCHUNK:
            gcp[nxt] = gather(nxt)
        gcp[i].wait()
        scp[i] = store(i)
    for i in range(NCHUNK - skew, NCHUNK):
        scp[i].wait()


@jax.jit
def _gather(pos, pos_embd):
    mesh = plsc.VectorSubcoreMesh(core_axis_name="c", subcore_axis_name="s")
    run = pl.kernel(
        _gather_body,
        mesh=mesh,
        out_type=jax.ShapeDtypeStruct((B, D), jnp.float32),
        scratch_types=[
            pltpu.VMEM((B_PER_W,), jnp.int32),
            pltpu.VMEM((NBUF, CHUNK, D), jnp.float32),
            pltpu.SemaphoreType.DMA((NBUF,)),
            pltpu.SemaphoreType.DMA((NBUF,)),
        ],
    )
    return run(pos_embd, pos)


def kernel(pos, pos_embd):
    return _gather(pos, pos_embd)
